# Initial kernel scaffold; baseline (speedup 1.0000x reference)
import jax
import jax.numpy as jnp
from jax.experimental import pallas as pl


def kernel(ed, threshold_offsets):
    raise NotImplementedError("write your pallas kernel here")


# speedup vs baseline: 1.7923x; 1.7923x over previous
import functools

import numpy as np
import jax
import jax.numpy as jnp
from jax import lax
from jax.experimental import pallas as pl
from jax.experimental.pallas import tpu as pltpu
from jax.experimental.pallas import tpu_sc as plsc

TEMPERATURE = 8.0
OFFSET_SCALE = 0.2
MIN_GAP = 0.001

NC = 2
NS = 16
NW = NC * NS
LANES = 16
RADIX = 256
NCHAIN = 6

_MESH = dict(core_axis_name="c", subcore_axis_name="s")


def _selection_constants(n):
  fr = np.array([0.55, 0.25, 0.15, 0.05], np.float32)
  s = np.float32(fr.sum())
  frn = (fr / np.maximum(s, np.float32(1e-8))).astype(np.float32)
  cdf = np.cumsum(frn).astype(np.float32)[:-1]
  nf = np.float32(n)
  pos = (cdf * (nf - np.float32(1))).astype(np.float32)
  low = np.floor(pos)
  high = np.ceil(pos)
  hw = (pos - low).astype(np.float32)
  lw = (np.float32(1) - hw).astype(np.float32)
  lowc = np.clip(low, np.float32(0), nf - np.float32(1)).astype(np.int64)
  highc = np.clip(high, np.float32(0), nf - np.float32(1)).astype(np.int64)
  ranks = []
  for i in range(3):
    ranks += [int(lowc[i]), int(highc[i])]
  return ranks, [float(lw[i]) for i in range(3)], [float(hw[i]) for i in range(3)]


def _iota16():
  return lax.iota(jnp.int32, 16)


def _extract_i32(vec, lane):
  return jnp.sum(jnp.where(_iota16() == lane, vec, jnp.int32(0)))


def _extract_u32(vec_u, lane):
  return jnp.sum(jnp.where(_iota16() == lane, vec_u, jnp.uint32(0)))


def _extract_f32(vec, lane):
  return jnp.sum(jnp.where(_iota16() == lane, vec, jnp.float32(0)))


def _monotone_key(x):
  bi = lax.bitcast_convert_type(x, jnp.int32)
  bu = lax.bitcast_convert_type(x, jnp.uint32)
  return jnp.where(bi < 0, ~bu, bu | jnp.uint32(0x80000000))


def _key_to_f32(key_u):
  ki = lax.bitcast_convert_type(key_u, jnp.int32)
  bits = jnp.where(ki < 0, key_u & jnp.uint32(0x7FFFFFFF), ~key_u)
  return lax.bitcast_convert_type(bits, jnp.float32)


def _zero_ref(ref, words):
  z = jnp.zeros((16,), jnp.int32)

  @pl.loop(0, words // 16)
  def _(j):
    ref[pl.ds(j * 16, 16)] = z


def _merge_hists(hin, merged, words):
  @pl.loop(0, words // 16)
  def _(j):
    acc = jnp.zeros((16,), jnp.int32)
    for w in range(NW):
      acc = acc + hin[pl.ds(w * words + j * 16, 16)]
    merged[pl.ds(j * 16, 16)] = acc


def _select_digit(merged, region_off, k):
  def body(j, carry):
    run, dcnt, below = carry
    h = merged[pl.ds(region_off + j * 16, 16)]
    c = plsc.cumsum(h) + run
    m = c <= k
    run = run + jnp.sum(h)
    dcnt = dcnt + jnp.sum(jnp.where(m, jnp.int32(1), jnp.int32(0)))
    below = below + jnp.sum(jnp.where(m, h, jnp.int32(0)))
    return run, dcnt, below

  z = jnp.int32(0)
  _, dcnt, below = pl.loop(0, 16, init_carry=(z, z, z))(body)
  return dcnt, below


def _scan_pass(ed, buf, hist, wid, per, ch, shift, prefixes):
  nchain = 1 if prefixes is None else NCHAIN
  _zero_ref(hist, nchain * LANES * RADIX)
  lane_off = _iota16() * RADIX
  ones = jnp.ones((16,), jnp.int32)
  base = wid * per

  @pl.loop(0, per // ch)
  def _(c):
    pltpu.sync_copy(ed.at[pl.ds(base + c * ch, ch)], buf)

    @pl.loop(0, ch // 16)
    def _(i):
      x = buf[pl.ds(i * 16, 16)]
      key = _monotone_key(x)
      dig = ((key >> shift) & jnp.uint32(0xFF)).astype(jnp.int32)
      idx0 = lane_off + dig
      if prefixes is None:
        plsc.addupdate_scatter(hist, [idx0], ones)
      else:
        khi = key >> (shift + 8)
        for r in range(NCHAIN):
          m = khi == prefixes[r]
          plsc.addupdate_scatter(hist, [idx0 + r * (LANES * RADIX)], ones,
                                 mask=m)


def _reduce_lanes(hist, local, nchain):
  @pl.loop(0, nchain * RADIX // 16)
  def _(j):
    chain = j // 16
    b0 = (j % 16) * 16
    acc = jnp.zeros((16,), jnp.int32)
    for l in range(LANES):
      acc = acc + hist[pl.ds(chain * (LANES * RADIX) + l * RADIX + b0, 16)]
    local[pl.ds(j * 16, 16)] = acc


def _load_state(state_hbm, statev):
  pltpu.sync_copy(state_hbm, statev)
  sv = statev[...]
  svu = lax.bitcast_convert_type(sv, jnp.uint32)
  prefixes = [_extract_u32(svu, r) for r in range(NCHAIN)]
  ks = [_extract_i32(sv, 8 + r) for r in range(NCHAIN)]
  return prefixes, ks


def _store_state(prefixes, ks, statev, state_out, wid):
  vec = jnp.zeros((16,), jnp.int32)
  it = _iota16()
  for r in range(NCHAIN):
    pi = lax.bitcast_convert_type(prefixes[r], jnp.int32)
    vec = jnp.where(it == r, pi, vec)
    vec = jnp.where(it == 8 + r, ks[r], vec)
  statev[...] = vec

  @pl.when(wid == 0)
  def _():
    pltpu.sync_copy(statev, state_out)


def _advance(merged, prefixes, ks, shared_region):
  new_p, new_k = [], []
  for r in range(NCHAIN):
    off = 0 if shared_region else r * RADIX
    d, below = _select_digit(merged, off, ks[r])
    du = lax.bitcast_convert_type(d, jnp.uint32) & jnp.uint32(0xFF)
    new_p.append((prefixes[r] << 8) | du)
    new_k.append(ks[r] - below)
  return new_p, new_k


def _wid():
  return lax.axis_index("c") * NS + lax.axis_index("s")


@functools.cache
def _build(n):
  assert n % NW == 0
  per = n // NW
  ch = 12800
  assert per % ch == 0
  ranks, lws, hws = _selection_constants(n)
  mesh = plsc.VectorSubcoreMesh(num_cores=NC, num_subcores=NS, **_MESH)

  @functools.partial(
      pl.kernel,
      out_type=jax.ShapeDtypeStruct((NW * RADIX,), jnp.int32),
      mesh=mesh,
      compiler_params=pltpu.CompilerParams(needs_layout_passes=False),
      scratch_types=[
          pltpu.VMEM((LANES * RADIX,), jnp.int32),
          pltpu.VMEM((RADIX,), jnp.int32),
          pltpu.VMEM((ch,), jnp.float32),
      ],
  )
  def k0(ed, hists_out, hist, local, buf):
    wid = _wid()
    _scan_pass(ed, buf, hist, wid, per, ch, 24, None)
    _reduce_lanes(hist, local, 1)
    pltpu.sync_copy(local, hists_out.at[pl.ds(wid * RADIX, RADIX)])

  def make_pass(p):
    shift = 24 - 8 * p
    chains_prev = 1 if p == 1 else NCHAIN
    prev_words = chains_prev * RADIX

    scratch = [
        pltpu.VMEM((NW * prev_words,), jnp.int32),
        pltpu.VMEM((prev_words,), jnp.int32),
        pltpu.VMEM((NCHAIN * LANES * RADIX,), jnp.int32),
        pltpu.VMEM((NCHAIN * RADIX,), jnp.int32),
        pltpu.VMEM((ch,), jnp.float32),
        pltpu.VMEM((16,), jnp.int32),
    ]
    out_type = (
        jax.ShapeDtypeStruct((NW * NCHAIN * RADIX,), jnp.int32),
        jax.ShapeDtypeStruct((16,), jnp.int32),
    )

    if p == 1:
      @functools.partial(pl.kernel, out_type=out_type, mesh=mesh,
      compiler_params=pltpu.CompilerParams(needs_layout_passes=False),
                         scratch_types=scratch)
      def kp(ed, hprev, hists_out, state_out,
             hin, merged, hist, local, buf, statev):
        wid = _wid()
        pltpu.sync_copy(hprev, hin)
        _merge_hists(hin, merged, prev_words)
        prefixes = [jnp.uint32(0)] * NCHAIN
        ks = [jnp.int32(ranks[r]) for r in range(NCHAIN)]
        prefixes, ks = _advance(merged, prefixes, ks, True)
        _store_state(prefixes, ks, statev, state_out, wid)
        _scan_pass(ed, buf, hist, wid, per, ch, shift, prefixes)
        _reduce_lanes(hist, local, NCHAIN)
        pltpu.sync_copy(
            local, hists_out.at[pl.ds(wid * NCHAIN * RADIX, NCHAIN * RADIX)])
    else:
      @functools.partial(pl.kernel, out_type=out_type, mesh=mesh,
      compiler_params=pltpu.CompilerParams(needs_layout_passes=False),
                         scratch_types=scratch)
      def kp(ed, hprev, sprev, hists_out, state_out,
             hin, merged, hist, local, buf, statev):
        wid = _wid()
        pltpu.sync_copy(hprev, hin)
        _merge_hists(hin, merged, prev_words)
        prefixes, ks = _load_state(sprev, statev)
        prefixes, ks = _advance(merged, prefixes, ks, False)
        _store_state(prefixes, ks, statev, state_out, wid)
        _scan_pass(ed, buf, hist, wid, per, ch, shift, prefixes)
        _reduce_lanes(hist, local, NCHAIN)
        pltpu.sync_copy(
            local, hists_out.at[pl.ds(wid * NCHAIN * RADIX, NCHAIN * RADIX)])

    return kp

  k1 = make_pass(1)
  k2 = make_pass(2)
  k3 = make_pass(3)

  chew = 6400
  assert per % chew == 0

  @functools.partial(
      pl.kernel,
      out_type=(
          jax.ShapeDtypeStruct((n * 4,), jnp.float32),
          jax.ShapeDtypeStruct((n,), jnp.int32),
          jax.ShapeDtypeStruct((16,), jnp.float32),
      ),
      mesh=mesh,
      compiler_params=pltpu.CompilerParams(needs_layout_passes=False),
      scratch_types=[
          pltpu.VMEM((NW * NCHAIN * RADIX,), jnp.int32),
          pltpu.VMEM((NCHAIN * RADIX,), jnp.int32),
          pltpu.VMEM((16,), jnp.int32),
          pltpu.VMEM((16,), jnp.float32),
          pltpu.VMEM((16,), jnp.float32),
          pltpu.VMEM((chew,), jnp.float32),
          pltpu.VMEM((chew * 4,), jnp.float32),
          pltpu.VMEM((chew,), jnp.int32),
      ],
  )
  def kew(ed, hprev, sprev, offs, probs_out, route_out, thr_out,
          hin, merged, statev, offv, thrv, xbuf, pbuf, rbuf):
    wid = _wid()
    pltpu.sync_copy(hprev, hin)
    _merge_hists(hin, merged, NCHAIN * RADIX)
    prefixes, ks = _load_state(sprev, statev)
    prefixes, _ = _advance(merged, prefixes, ks, False)

    it = _iota16()
    keyv = jnp.zeros((16,), jnp.uint32)
    for r in range(NCHAIN):
      keyv = jnp.where(it == r, prefixes[r], keyv)
    vals = _key_to_f32(keyv)
    v = [_extract_f32(vals, r) for r in range(NCHAIN)]

    pltpu.sync_copy(offs, offv)
    ov = offv[...]
    off = [_extract_f32(ov, i) for i in range(3)]

    base = [v[2 * i] * jnp.float32(lws[i]) + v[2 * i + 1] * jnp.float32(hws[i])
            for i in range(3)]
    raw = [base[i] + off[i] for i in range(3)]
    t1 = raw[0]
    t2 = jnp.maximum(raw[1], t1 + jnp.float32(MIN_GAP))
    t3 = jnp.maximum(raw[2], t2 + jnp.float32(MIN_GAP))
    lww = jnp.maximum(t2 - t1, jnp.float32(0.001))
    rww = jnp.maximum(t3 - t2, jnp.float32(0.001))
    half = jnp.float32(0.5)
    cen = [t1 - lww, (t1 + t2) * half, (t2 + t3) * half, t3 + rww]

    tv = jnp.zeros((16,), jnp.float32)
    tv = jnp.where(it == 0, t1, tv)
    tv = jnp.where(it == 1, t2, tv)
    tv = jnp.where(it == 2, t3, tv)
    thrv[...] = tv

    @pl.when(wid == 0)
    def _():
      pltpu.sync_copy(thrv, thr_out)

    lane4 = it * 4
    neg_t = jnp.float32(-TEMPERATURE)
    one_i = jnp.int32(1)
    zero_i = jnp.int32(0)
    base_e = wid * per

    @pl.loop(0, per // chew)
    def _(c):
      pltpu.sync_copy(ed.at[pl.ds(base_e + c * chew, chew)], xbuf)

      @pl.loop(0, chew // 16)
      def _(i):
        x = xbuf[pl.ds(i * 16, 16)]
        l0 = neg_t * jnp.abs(x - cen[0])
        l1 = neg_t * jnp.abs(x - cen[1])
        l2 = neg_t * jnp.abs(x - cen[2])
        l3 = neg_t * jnp.abs(x - cen[3])
        m = jnp.maximum(jnp.maximum(l0, l1), jnp.maximum(l2, l3))
        e0 = jnp.exp(l0 - m)
        e1 = jnp.exp(l1 - m)
        e2 = jnp.exp(l2 - m)
        e3 = jnp.exp(l3 - m)
        inv = jnp.float32(1.0) / (((e0 + e1) + e2) + e3)
        ib = i * 64
        plsc.store_scatter(pbuf, [lane4 + ib], e0 * inv)
        plsc.store_scatter(pbuf, [lane4 + (ib + 1)], e1 * inv)
        plsc.store_scatter(pbuf, [lane4 + (ib + 2)], e2 * inv)
        plsc.store_scatter(pbuf, [lane4 + (ib + 3)], e3 * inv)
        rt = (jnp.where(x > t1, one_i, zero_i)
              + jnp.where(x > t2, one_i, zero_i)
              + jnp.where(x > t3, one_i, zero_i))
        rbuf[pl.ds(i * 16, 16)] = rt

      pltpu.sync_copy(pbuf,
                      probs_out.at[pl.ds((base_e + c * chew) * 4, chew * 4)])
      pltpu.sync_copy(rbuf, route_out.at[pl.ds(base_e + c * chew, chew)])

  return k0, k1, k2, k3, kew


def kernel(ed, threshold_offsets):
  n = ed.size
  k0, k1, k2, k3, kew = _build(n)
  ed_flat = ed.reshape(-1)
  offs = (OFFSET_SCALE * jnp.tanh(threshold_offsets)).astype(jnp.float32)
  offs16 = jnp.zeros((16,), jnp.float32).at[:3].set(offs)

  h0 = k0(ed_flat)
  h1, s1 = k1(ed_flat, h0)
  h2, s2 = k2(ed_flat, h1, s1)
  h3, s3 = k3(ed_flat, h2, s2)
  probs_flat, route_flat, thr16 = kew(ed_flat, h3, s3, offs16)

  probs = probs_flat.reshape(ed.shape + (4,))
  route = route_flat.reshape(ed.shape)
  thresholds = thr16[:3]
  return route, probs, thresholds
